# baseline (speedup 1.0000x reference)
import functools

import jax
import jax.numpy as jnp
from jax import lax
from jax.experimental import pallas as pl
from jax.experimental.pallas import tpu as pltpu
from jax.experimental.pallas import tpu_sc as plsc

D = 768
LANES = 16
ROW_V = D // LANES

NC = 2
NS = 16
NW = NC * NS
PCHUNK = 8
NBUF = 3


def _make_emb_kernel(B: int, S: int):
    BS = B * S
    pos_w = S // NW
    nt = pos_w // PCHUNK
    chunk = B * PCHUNK
    per_w = BS // NW

    mesh = plsc.VectorSubcoreMesh(core_axis_name="c", subcore_axis_name="s")

    scratch = [pltpu.VMEM((nt, chunk), jnp.int32)]
    scratch += [pltpu.VMEM((chunk, D), jnp.float32) for _ in range(NBUF)]
    scratch += [pltpu.VMEM((pos_w, D), jnp.float32)]
    scratch += [pltpu.SemaphoreType.DMA for _ in range(2 * NBUF + 1)]

    @functools.partial(
        pl.kernel,
        mesh=mesh,
        out_type=jax.ShapeDtypeStruct((BS, D), jnp.float32),
        scratch_types=scratch,
    )
    def emb_kernel(ids_hbm, wte_hbm, wpe_hbm,
                   out_hbm, idx_v, *rest):
        bufs = rest[:NBUF]
        wpe_v = rest[NBUF]
        sem_g = rest[NBUF + 1:NBUF + 1 + NBUF]
        sem_o = rest[NBUF + 1 + NBUF:NBUF + 1 + 2 * NBUF]
        sem_w = rest[NBUF + 1 + 2 * NBUF]

        wid = lax.axis_index("s") * NC + lax.axis_index("c")
        pbase = wid * pos_w

        idx_cps = [
            pltpu.async_copy(
                ids_hbm.at[pl.ds(b * S + pbase + t * PCHUNK, PCHUNK)],
                idx_v.at[t, pl.ds(b * PCHUNK, PCHUNK)],
                sem_w)
            for t in range(nt) for b in range(B)
        ]
        for cp in idx_cps:
            cp.wait()

        gt = [None] * nt
        out_cp = [[] for _ in range(NBUF)]
        for t in range(NBUF - 1):
            gt[t] = pltpu.async_copy(
                wte_hbm.at[idx_v.at[t]], bufs[t % NBUF], sem_g[t % NBUF])

        pltpu.async_copy(
            wpe_hbm.at[pl.ds(pbase, pos_w)], wpe_v, sem_w).wait()

        for t in range(nt + NBUF - 1):
            if NBUF - 1 <= t < nt:
                p = t % NBUF
                for cp in out_cp[p]:
                    cp.wait()
                out_cp[p] = []
                gt[t] = pltpu.async_copy(
                    wte_hbm.at[idx_v.at[t]], bufs[p], sem_g[p])
            u = t - (NBUF - 1)
            if 0 <= u < nt:
                p = u % NBUF
                gt[u].wait()
                buf = bufs[p]

                out_cp[p] = []
                for b in range(B):
                    @plsc.parallel_loop(0, PCHUNK, unroll=1)
                    def add_pos(r):
                        for j in range(ROW_V):
                            sl = pl.ds(j * LANES, LANES)
                            plsc.addupdate(buf.at[b * PCHUNK + r, sl],
                                           wpe_v[u * PCHUNK + r, sl])

                    out_cp[p].append(pltpu.async_copy(
                        buf.at[pl.ds(b * PCHUNK, PCHUNK)],
                        out_hbm.at[pl.ds(b * S + pbase + u * PCHUNK,
                                         PCHUNK)],
                        sem_o[p]))
        for p in range(NBUF):
            for cp in out_cp[p]:
                cp.wait()

    return emb_kernel


def kernel(input_ids, attention_mask, wte, wpe):
    input_shape = input_ids.shape
    S = input_shape[-1]
    ids2 = input_ids.reshape(-1, S)
    B = ids2.shape[0]
    BS = B * S

    ids_flat = ids2.reshape(BS).astype(jnp.int32)
    mask2 = attention_mask.reshape(B, S).astype(jnp.float32)

    hidden = _make_emb_kernel(B, S)(ids_flat, wte, wpe)
    am = pl.pallas_call(
        _am_body,
        out_shape=jax.ShapeDtypeStruct((B, S), jnp.float32),
    )(mask2)
    hidden = hidden.reshape(B, S, D)
    am = am.reshape(B, 1, 1, S)
    return (hidden, am)


def _am_body(mask_ref, am_ref):
    am_ref[...] = (1.0 - mask_ref[...]) * jnp.finfo(jnp.float32).min


# speedup vs baseline: 1.1081x; 1.1081x over previous
import functools

import jax
import jax.numpy as jnp
from jax import lax
from jax.experimental import pallas as pl
from jax.experimental.pallas import tpu as pltpu
from jax.experimental.pallas import tpu_sc as plsc

D = 768
LANES = 16
ROW_V = D // LANES

NC = 2
NS = 16
NW = NC * NS
PCHUNK = 8
NBUF = 4


def _make_emb_kernel(B: int, S: int):
    BS = B * S
    pos_w = S // NW
    nt = pos_w // PCHUNK
    chunk = B * PCHUNK
    per_w = BS // NW

    mesh = plsc.VectorSubcoreMesh(core_axis_name="c", subcore_axis_name="s")

    scratch = [pltpu.VMEM((nt, chunk), jnp.int32)]
    scratch += [pltpu.VMEM((chunk, D), jnp.float32) for _ in range(NBUF)]
    scratch += [pltpu.VMEM((PCHUNK, D), jnp.float32)
                for _ in range(NBUF)]
    scratch += [pltpu.SemaphoreType.DMA for _ in range(3 * NBUF + 1)]

    @functools.partial(
        pl.kernel,
        mesh=mesh,
        out_type=jax.ShapeDtypeStruct((BS, D), jnp.float32),
        scratch_types=scratch,
    )
    def emb_kernel(ids_hbm, wte_hbm, wpe_hbm,
                   out_hbm, idx_v, *rest):
        bufs = rest[:NBUF]
        wring = rest[NBUF:2 * NBUF]
        sem_g = rest[2 * NBUF:3 * NBUF]
        sem_o = rest[3 * NBUF:4 * NBUF]
        sem_p = rest[4 * NBUF:5 * NBUF]
        sem_w = rest[5 * NBUF]

        wid = lax.axis_index("s") * NC + lax.axis_index("c")
        pbase = wid * pos_w

        idx_cps = [
            pltpu.async_copy(
                ids_hbm.at[pl.ds(b * S + pbase + t * PCHUNK, PCHUNK)],
                idx_v.at[t, pl.ds(b * PCHUNK, PCHUNK)],
                sem_w)
            for t in range(nt) for b in range(B)
        ]
        for cp in idx_cps:
            cp.wait()

        gt = [None] * nt
        wt = [None] * nt
        out_cp = [[] for _ in range(NBUF)]
        for t in range(NBUF - 1):
            gt[t] = pltpu.async_copy(
                wte_hbm.at[idx_v.at[t]], bufs[t % NBUF], sem_g[t % NBUF])
            wt[t] = pltpu.async_copy(
                wpe_hbm.at[pl.ds(pbase + t * PCHUNK, PCHUNK)],
                wring[t % NBUF], sem_p[t % NBUF])

        for t in range(nt + NBUF - 1):
            if NBUF - 1 <= t < nt:
                p = t % NBUF
                for cp in out_cp[p]:
                    cp.wait()
                out_cp[p] = []
                gt[t] = pltpu.async_copy(
                    wte_hbm.at[idx_v.at[t]], bufs[p], sem_g[p])
                wt[t] = pltpu.async_copy(
                    wpe_hbm.at[pl.ds(pbase + t * PCHUNK, PCHUNK)],
                    wring[p], sem_p[p])
            u = t - (NBUF - 1)
            if 0 <= u < nt:
                p = u % NBUF
                gt[u].wait()
                wt[u].wait()
                buf = bufs[p]
                wv = wring[p]

                @plsc.parallel_loop(0, PCHUNK, unroll=1)
                def add_pos(r):
                    for j in range(ROW_V):
                        sl = pl.ds(j * LANES, LANES)
                        w = wv[r, sl]
                        for b in range(B):
                            plsc.addupdate(buf.at[b * PCHUNK + r, sl], w)

                out_cp[p] = [
                    pltpu.async_copy(
                        buf.at[pl.ds(b * PCHUNK, PCHUNK)],
                        out_hbm.at[pl.ds(b * S + pbase + u * PCHUNK,
                                         PCHUNK)],
                        sem_o[p])
                    for b in range(B)
                ]
        for p in range(NBUF):
            for cp in out_cp[p]:
                cp.wait()

    return emb_kernel


def kernel(input_ids, attention_mask, wte, wpe):
    input_shape = input_ids.shape
    S = input_shape[-1]
    ids2 = input_ids.reshape(-1, S)
    B = ids2.shape[0]
    BS = B * S

    ids_flat = ids2.reshape(BS).astype(jnp.int32)
    mask2 = attention_mask.reshape(B, S).astype(jnp.float32)

    hidden = _make_emb_kernel(B, S)(ids_flat, wte, wpe)
    am = pl.pallas_call(
        _am_body,
        out_shape=jax.ShapeDtypeStruct((B, S), jnp.float32),
    )(mask2)
    hidden = hidden.reshape(B, S, D)
    am = am.reshape(B, 1, 1, S)
    return (hidden, am)


def _am_body(mask_ref, am_ref):
    am_ref[...] = (1.0 - mask_ref[...]) * jnp.finfo(jnp.float32).min
